# baseline (speedup 1.0000x reference)
import functools

import jax
import jax.numpy as jnp
from jax import lax
from jax.experimental import pallas as pl
from jax.experimental.pallas import tpu as pltpu
from jax.experimental.pallas import tpu_sc as plsc

E = 16
TOP_K = 2
H = 768
I = 256
T = 2048
TB = 256
NB = (TOP_K * T) // TB + E - 1
SLOTS = NB * TB
NTILES = 32
TPT = T // NTILES
SPT = SLOTS // NTILES


def _k1_body(x_ref, wr_ref, d0_ref, d1_ref, wt0_ref, wt1_ref, be_ref):
    x = x_ref[...]
    logits = lax.dot_general(x, wr_ref[...], (((1,), (1,)), ((), ())),
                             preferred_element_type=jnp.float32)
    scores = jax.nn.sigmoid(logits)
    eidx = lax.broadcasted_iota(jnp.int32, scores.shape, 1)
    m1 = jnp.max(scores, axis=1, keepdims=True)
    i1 = jnp.min(jnp.where(scores == m1, eidx, E), axis=1, keepdims=True)
    excl = eidx == i1
    s2 = jnp.where(excl, -jnp.inf, scores)
    m2 = jnp.max(s2, axis=1, keepdims=True)
    i2 = jnp.min(jnp.where(s2 == m2, eidx, E), axis=1, keepdims=True)
    denom = m1 + m2 + 1e-20
    w1 = m1 / denom
    w2 = m2 / denom

    oh1 = (eidx == i1).astype(jnp.float32)
    oh2 = (eidx == i2).astype(jnp.float32)
    oh = oh1 + oh2
    r_io = lax.broadcasted_iota(jnp.int32, (T, T), 0)
    c_io = lax.broadcasted_iota(jnp.int32, (T, T), 1)
    tri = (c_io < r_io).astype(jnp.float32)
    c = lax.dot_general(tri, oh, (((1,), (0,)), ((), ())),
                        preferred_element_type=jnp.float32)
    counts = jnp.sum(oh, axis=0, keepdims=True)
    nb = jnp.floor((counts + (TB - 1)) / TB)
    e_io0 = lax.broadcasted_iota(jnp.int32, (E, E), 0)
    e_io1 = lax.broadcasted_iota(jnp.int32, (E, E), 1)
    triu = (e_io0 < e_io1).astype(jnp.float32)
    base_block = lax.dot_general(nb, triu, (((1,), (0,)), ((), ())),
                                 preferred_element_type=jnp.float32)
    slot_base = TB * base_block
    vals = slot_base + c
    dest0 = jnp.sum(oh1 * vals, axis=1, keepdims=True)
    dest1 = jnp.sum(oh2 * vals, axis=1, keepdims=True)
    d0_ref[...] = jnp.transpose(dest0.astype(jnp.int32)).reshape(T)
    d1_ref[...] = jnp.transpose(dest1.astype(jnp.int32)).reshape(T)
    wt0_ref[...] = jnp.transpose(w1).reshape(T)
    wt1_ref[...] = jnp.transpose(w2).reshape(T)

    ends = base_block + nb
    jrow = lax.broadcasted_iota(jnp.int32, (1, 128), 1).astype(jnp.float32)
    cmp = (jnp.transpose(base_block) <= jrow).astype(jnp.float32)
    bexp = jnp.sum(cmp, axis=0, keepdims=True) - 1.0
    num_active = jnp.sum(nb, axis=1, keepdims=True)
    lane = lax.broadcasted_iota(jnp.int32, (1, 128), 1)
    meta = jnp.where(lane == 127, num_active, bexp)
    be_ref[...] = meta.astype(jnp.int32).reshape(128)


def _k1(x, W_router):
    return pl.pallas_call(
        _k1_body,
        out_shape=(
            jax.ShapeDtypeStruct((T,), jnp.int32),
            jax.ShapeDtypeStruct((T,), jnp.int32),
            jax.ShapeDtypeStruct((T,), jnp.float32),
            jax.ShapeDtypeStruct((T,), jnp.float32),
            jax.ShapeDtypeStruct((128,), jnp.int32),
        ),
    )(x, W_router)


def _k2_body(x_hbm, d0_hbm, d1_hbm, xs_hbm,
             d0_v, d1_v, table_v, rows_v, sem):
    wid = lax.axis_index("s") * 2 + lax.axis_index("c")
    pltpu.sync_copy(d0_hbm, d0_v)
    pltpu.sync_copy(d1_hbm, d1_v)

    def _zero(i, _):
        table_v[pl.ds(i * 16, 16)] = jnp.zeros((16,), jnp.int32)
        return 0
    lax.fori_loop(0, SLOTS // 16, _zero, 0)

    def _scat(i, _):
        toks = lax.iota(jnp.int32, 16) + i * 16
        plsc.store_scatter(table_v, [d0_v[pl.ds(i * 16, 16)]], toks)
        plsc.store_scatter(table_v, [d1_v[pl.ds(i * 16, 16)]], toks)
        return 0
    lax.fori_loop(0, T // 16, _scat, 0)

    base = wid * SPT
    for c in range(4):
        off = c * 64
        n = 64 if c < 3 else SPT - 3 * 64
        idx = table_v.at[pl.ds(base + off, n)]
        pltpu.async_copy(x_hbm.at[idx], rows_v.at[pl.ds(0, n)], sem).wait()
        pltpu.sync_copy(rows_v.at[pl.ds(0, n)],
                        xs_hbm.at[pl.ds(base + off, n)])


def _k2(x, dest0, dest1):
    mesh = plsc.VectorSubcoreMesh(core_axis_name="c", subcore_axis_name="s")
    return pl.kernel(
        _k2_body,
        mesh=mesh,
        compiler_params=pltpu.CompilerParams(needs_layout_passes=False),
        out_type=jax.ShapeDtypeStruct((SLOTS, H), jnp.float32),
        scratch_types=[
            pltpu.VMEM((T,), jnp.int32),
            pltpu.VMEM((T,), jnp.int32),
            pltpu.VMEM((SLOTS,), jnp.int32),
            pltpu.VMEM((64, H), jnp.float32),
            pltpu.SemaphoreType.DMA,
        ],
    )(x, dest0, dest1)


def _k3_body(be_ref, xs_ref, wg_ref, wu_ref, wd_ref, ys_ref):
    j = pl.program_id(0)
    na = be_ref[127]

    @pl.when(j < na)
    def _mlp():
        xs = xs_ref[...]
        g = lax.dot_general(xs, wg_ref[0], (((1,), (0,)), ((), ())),
                            preferred_element_type=jnp.float32)
        u = lax.dot_general(xs, wu_ref[0], (((1,), (0,)), ((), ())),
                            preferred_element_type=jnp.float32)
        hmid = g * jax.nn.sigmoid(g) * u
        ys_ref[...] = lax.dot_general(hmid, wd_ref[0], (((1,), (0,)), ((), ())),
                                      preferred_element_type=jnp.float32)


def _k3(xs, W_gate, W_up, W_down, be):
    grid_spec = pltpu.PrefetchScalarGridSpec(
        num_scalar_prefetch=1,
        grid=(NB,),
        in_specs=[
            pl.BlockSpec((TB, H), lambda j, be: (j, 0)),
            pl.BlockSpec((1, H, I), lambda j, be: (be[j], 0, 0)),
            pl.BlockSpec((1, H, I), lambda j, be: (be[j], 0, 0)),
            pl.BlockSpec((1, I, H), lambda j, be: (be[j], 0, 0)),
        ],
        out_specs=pl.BlockSpec((TB, H), lambda j, be: (j, 0)),
    )
    return pl.pallas_call(
        _k3_body,
        grid_spec=grid_spec,
        out_shape=jax.ShapeDtypeStruct((SLOTS, H), jnp.float32),
    )(be, xs, W_gate, W_up, W_down)


def _k4_body(ys_hbm, d0_hbm, d1_hbm, wt0_hbm, wt1_hbm, out_hbm,
             d0_v, d1_v, w0_v, w1_v, rows0_v, rows1_v, sem):
    wid = lax.axis_index("s") * 2 + lax.axis_index("c")
    base = wid * TPT
    pltpu.sync_copy(d0_hbm.at[pl.ds(base, TPT)], d0_v)
    pltpu.sync_copy(d1_hbm.at[pl.ds(base, TPT)], d1_v)
    pltpu.sync_copy(wt0_hbm.at[pl.ds(base, TPT)], w0_v)
    pltpu.sync_copy(wt1_hbm.at[pl.ds(base, TPT)], w1_v)
    pltpu.async_copy(ys_hbm.at[d0_v], rows0_v, sem).wait()
    pltpu.async_copy(ys_hbm.at[d1_v], rows1_v, sem).wait()

    def _tok(r, _):
        rsplat = jnp.full((16,), 0, jnp.int32) + r
        wa = plsc.load_gather(w0_v, [rsplat])
        wb = plsc.load_gather(w1_v, [rsplat])
        def _chunk(cidx, _):
            a = rows0_v[r, pl.ds(cidx * 16, 16)]
            b = rows1_v[r, pl.ds(cidx * 16, 16)]
            rows0_v[r, pl.ds(cidx * 16, 16)] = a * wa + b * wb
            return 0
        lax.fori_loop(0, H // 16, _chunk, 0)
        return 0
    lax.fori_loop(0, TPT, _tok, 0)
    pltpu.sync_copy(rows0_v, out_hbm.at[pl.ds(base, TPT)])


def _k4(ys, dest0, dest1, wt0, wt1):
    mesh = plsc.VectorSubcoreMesh(core_axis_name="c", subcore_axis_name="s")
    return pl.kernel(
        _k4_body,
        mesh=mesh,
        compiler_params=pltpu.CompilerParams(needs_layout_passes=False),
        out_type=jax.ShapeDtypeStruct((T, H), jnp.float32),
        scratch_types=[
            pltpu.VMEM((TPT,), jnp.int32),
            pltpu.VMEM((TPT,), jnp.int32),
            pltpu.VMEM((TPT,), jnp.float32),
            pltpu.VMEM((TPT,), jnp.float32),
            pltpu.VMEM((TPT, H), jnp.float32),
            pltpu.VMEM((TPT, H), jnp.float32),
            pltpu.SemaphoreType.DMA,
        ],
    )(ys, dest0, dest1, wt0, wt1)


def kernel(hidden_states, W_router, W_gate, W_up, W_down):
    b, s, h = hidden_states.shape
    x = hidden_states.reshape(T, h)
    dest0, dest1, wt0, wt1, be = _k1(x, W_router)
    xs = _k2(x, dest0, dest1)
    ys = _k3(xs, W_gate, W_up, W_down, be)
    out = _k4(ys, dest0, dest1, wt0, wt1)
    return out.reshape(b, s, h)


# speedup vs baseline: 2.7841x; 2.7841x over previous
import functools

import jax
import jax.numpy as jnp
from jax import lax
from jax.experimental import pallas as pl
from jax.experimental.pallas import tpu as pltpu
from jax.experimental.pallas import tpu_sc as plsc

E = 16
TOP_K = 2
H = 768
I = 256
T = 2048
TB = 256
NB = (TOP_K * T) // TB + E - 1
SLOTS = NB * TB
NTILES = 32
TPT = T // NTILES
SPT = SLOTS // NTILES


def _k1_body(x_ref, wr_ref, d0_ref, d1_ref, wt0_ref, wt1_ref, be_ref):
    x = x_ref[...]
    logits = lax.dot_general(x, wr_ref[...], (((1,), (1,)), ((), ())),
                             preferred_element_type=jnp.float32)
    scores = jax.nn.sigmoid(logits)
    eidx = lax.broadcasted_iota(jnp.int32, scores.shape, 1)
    m1 = jnp.max(scores, axis=1, keepdims=True)
    i1 = jnp.min(jnp.where(scores == m1, eidx, E), axis=1, keepdims=True)
    excl = eidx == i1
    s2 = jnp.where(excl, -jnp.inf, scores)
    m2 = jnp.max(s2, axis=1, keepdims=True)
    i2 = jnp.min(jnp.where(s2 == m2, eidx, E), axis=1, keepdims=True)
    denom = m1 + m2 + 1e-20
    w1 = m1 / denom
    w2 = m2 / denom

    oh1 = (eidx == i1).astype(jnp.float32)
    oh2 = (eidx == i2).astype(jnp.float32)
    oh = oh1 + oh2
    r_io = lax.broadcasted_iota(jnp.int32, (T, T), 0)
    c_io = lax.broadcasted_iota(jnp.int32, (T, T), 1)
    tri = (c_io < r_io).astype(jnp.float32)
    c = lax.dot_general(tri, oh, (((1,), (0,)), ((), ())),
                        preferred_element_type=jnp.float32)
    counts = jnp.sum(oh, axis=0, keepdims=True)
    nb = jnp.floor((counts + (TB - 1)) / TB)
    e_io0 = lax.broadcasted_iota(jnp.int32, (E, E), 0)
    e_io1 = lax.broadcasted_iota(jnp.int32, (E, E), 1)
    triu = (e_io0 < e_io1).astype(jnp.float32)
    base_block = lax.dot_general(nb, triu, (((1,), (0,)), ((), ())),
                                 preferred_element_type=jnp.float32)
    slot_base = TB * base_block
    vals = slot_base + c
    dest0 = jnp.sum(oh1 * vals, axis=1, keepdims=True)
    dest1 = jnp.sum(oh2 * vals, axis=1, keepdims=True)
    d0_ref[...] = jnp.transpose(dest0.astype(jnp.int32)).reshape(T)
    d1_ref[...] = jnp.transpose(dest1.astype(jnp.int32)).reshape(T)
    wt0_ref[...] = jnp.transpose(w1).reshape(T)
    wt1_ref[...] = jnp.transpose(w2).reshape(T)

    ends = base_block + nb
    jrow = lax.broadcasted_iota(jnp.int32, (1, 128), 1).astype(jnp.float32)
    cmp = (jnp.transpose(base_block) <= jrow).astype(jnp.float32)
    bexp = jnp.sum(cmp, axis=0, keepdims=True) - 1.0
    num_active = jnp.sum(nb, axis=1, keepdims=True)
    lane = lax.broadcasted_iota(jnp.int32, (1, 128), 1)
    meta = jnp.where(lane == 127, num_active, bexp)
    be_ref[...] = meta.astype(jnp.int32).reshape(128)


def _k1(x, W_router):
    return pl.pallas_call(
        _k1_body,
        out_shape=(
            jax.ShapeDtypeStruct((T,), jnp.int32),
            jax.ShapeDtypeStruct((T,), jnp.int32),
            jax.ShapeDtypeStruct((T,), jnp.float32),
            jax.ShapeDtypeStruct((T,), jnp.float32),
            jax.ShapeDtypeStruct((128,), jnp.int32),
        ),
    )(x, W_router)


def _k2_body(x_hbm, d0_hbm, d1_hbm, xs_hbm,
             d0_v, d1_v, rows_v, sem, sem2):
    wid = lax.axis_index("s") * 2 + lax.axis_index("c")
    base = wid * TPT
    pltpu.sync_copy(d0_hbm.at[pl.ds(base, TPT)], d0_v)
    pltpu.sync_copy(d1_hbm.at[pl.ds(base, TPT)], d1_v)
    pltpu.sync_copy(x_hbm.at[pl.ds(base, TPT)], rows_v)
    c0 = pltpu.async_copy(rows_v, xs_hbm.at[d0_v], sem)
    c1 = pltpu.async_copy(rows_v, xs_hbm.at[d1_v], sem2)
    c0.wait()
    c1.wait()


def _k2(x, dest0, dest1):
    mesh = plsc.VectorSubcoreMesh(core_axis_name="c", subcore_axis_name="s")
    return pl.kernel(
        _k2_body,
        mesh=mesh,
        compiler_params=pltpu.CompilerParams(needs_layout_passes=False),
        out_type=jax.ShapeDtypeStruct((SLOTS, H), jnp.float32),
        scratch_types=[
            pltpu.VMEM((TPT,), jnp.int32),
            pltpu.VMEM((TPT,), jnp.int32),
            pltpu.VMEM((TPT, H), jnp.float32),
            pltpu.SemaphoreType.DMA,
            pltpu.SemaphoreType.DMA,
        ],
    )(x, dest0, dest1)


def _k3_body(be_ref, xs_ref, wg_ref, wu_ref, wd_ref, ys_ref):
    j = pl.program_id(0)
    na = be_ref[127]

    @pl.when(j < na)
    def _mlp():
        xs = xs_ref[...]
        g = lax.dot_general(xs, wg_ref[0], (((1,), (0,)), ((), ())),
                            preferred_element_type=jnp.float32)
        u = lax.dot_general(xs, wu_ref[0], (((1,), (0,)), ((), ())),
                            preferred_element_type=jnp.float32)
        hmid = g * jax.nn.sigmoid(g) * u
        ys_ref[...] = lax.dot_general(hmid, wd_ref[0], (((1,), (0,)), ((), ())),
                                      preferred_element_type=jnp.float32)


def _k3(xs, W_gate, W_up, W_down, be):
    grid_spec = pltpu.PrefetchScalarGridSpec(
        num_scalar_prefetch=1,
        grid=(NB,),
        in_specs=[
            pl.BlockSpec((TB, H), lambda j, be: (j, 0)),
            pl.BlockSpec((1, H, I), lambda j, be: (be[j], 0, 0)),
            pl.BlockSpec((1, H, I), lambda j, be: (be[j], 0, 0)),
            pl.BlockSpec((1, I, H), lambda j, be: (be[j], 0, 0)),
        ],
        out_specs=pl.BlockSpec((TB, H), lambda j, be: (j, 0)),
    )
    return pl.pallas_call(
        _k3_body,
        grid_spec=grid_spec,
        out_shape=jax.ShapeDtypeStruct((SLOTS, H), jnp.float32),
    )(be, xs, W_gate, W_up, W_down)


def _k4_body(ys_hbm, d0_hbm, d1_hbm, wt0_hbm, wt1_hbm, out_hbm,
             d0_v, d1_v, w0_v, w1_v, rows0_v, rows1_v, sem, sem2):
    wid = lax.axis_index("s") * 2 + lax.axis_index("c")
    base = wid * TPT
    pltpu.sync_copy(d0_hbm.at[pl.ds(base, TPT)], d0_v)
    pltpu.sync_copy(d1_hbm.at[pl.ds(base, TPT)], d1_v)
    pltpu.sync_copy(wt0_hbm.at[pl.ds(base, TPT)], w0_v)
    pltpu.sync_copy(wt1_hbm.at[pl.ds(base, TPT)], w1_v)
    c0 = pltpu.async_copy(ys_hbm.at[d0_v], rows0_v, sem)
    c1 = pltpu.async_copy(ys_hbm.at[d1_v], rows1_v, sem2)
    c0.wait()
    c1.wait()

    def _tok(r, _):
        rsplat = jnp.full((16,), 0, jnp.int32) + r
        wa = plsc.load_gather(w0_v, [rsplat])
        wb = plsc.load_gather(w1_v, [rsplat])
        def _chunk(cidx, _):
            a = rows0_v[r, pl.ds(cidx * 16, 16)]
            b = rows1_v[r, pl.ds(cidx * 16, 16)]
            rows0_v[r, pl.ds(cidx * 16, 16)] = a * wa + b * wb
            return 0
        lax.fori_loop(0, H // 16, _chunk, 0, unroll=8)
        return 0
    lax.fori_loop(0, TPT, _tok, 0)
    pltpu.sync_copy(rows0_v, out_hbm.at[pl.ds(base, TPT)])


def _k4(ys, dest0, dest1, wt0, wt1):
    mesh = plsc.VectorSubcoreMesh(core_axis_name="c", subcore_axis_name="s")
    return pl.kernel(
        _k4_body,
        mesh=mesh,
        compiler_params=pltpu.CompilerParams(needs_layout_passes=False),
        out_type=jax.ShapeDtypeStruct((T, H), jnp.float32),
        scratch_types=[
            pltpu.VMEM((TPT,), jnp.int32),
            pltpu.VMEM((TPT,), jnp.int32),
            pltpu.VMEM((TPT,), jnp.float32),
            pltpu.VMEM((TPT,), jnp.float32),
            pltpu.VMEM((TPT, H), jnp.float32),
            pltpu.VMEM((TPT, H), jnp.float32),
            pltpu.SemaphoreType.DMA,
            pltpu.SemaphoreType.DMA,
        ],
    )(ys, dest0, dest1, wt0, wt1)


def kernel(hidden_states, W_router, W_gate, W_up, W_down):
    b, s, h = hidden_states.shape
    x = hidden_states.reshape(T, h)
    dest0, dest1, wt0, wt1, be = _k1(x, W_router)
    xs = _k2(x, dest0, dest1)
    ys = _k3(xs, W_gate, W_up, W_down, be)
    out = _k4(ys, dest0, dest1, wt0, wt1)
    return out.reshape(b, s, h)


# speedup vs baseline: 2.9550x; 1.0614x over previous
import functools

import jax
import jax.numpy as jnp
from jax import lax
from jax.experimental import pallas as pl
from jax.experimental.pallas import tpu as pltpu
from jax.experimental.pallas import tpu_sc as plsc

E = 16
TOP_K = 2
H = 768
I = 256
T = 2048
TB = 256
NB = (TOP_K * T) // TB + E
NB2 = NB // 2
SLOTS = NB * TB
SLOTS2 = SLOTS
NTILES = 32
TPT = T // NTILES
SPT = SLOTS // NTILES


def _k1_body(x_ref, wr_ref, d0_ref, d1_ref, wt0_ref, wt1_ref, be_ref):
    x = x_ref[...]
    logits = lax.dot_general(x, wr_ref[...], (((1,), (1,)), ((), ())),
                             preferred_element_type=jnp.float32)
    scores = jax.nn.sigmoid(logits)
    eidx = lax.broadcasted_iota(jnp.int32, scores.shape, 1)
    m1 = jnp.max(scores, axis=1, keepdims=True)
    i1 = jnp.min(jnp.where(scores == m1, eidx, E), axis=1, keepdims=True)
    excl = eidx == i1
    s2 = jnp.where(excl, -jnp.inf, scores)
    m2 = jnp.max(s2, axis=1, keepdims=True)
    i2 = jnp.min(jnp.where(s2 == m2, eidx, E), axis=1, keepdims=True)
    denom = m1 + m2 + 1e-20
    w1 = m1 / denom
    w2 = m2 / denom

    oh1 = (eidx == i1).astype(jnp.float32)
    oh2 = (eidx == i2).astype(jnp.float32)
    oh = oh1 + oh2
    r_io = lax.broadcasted_iota(jnp.int32, (T, T), 0)
    c_io = lax.broadcasted_iota(jnp.int32, (T, T), 1)
    tri = (c_io < r_io).astype(jnp.float32)
    c = lax.dot_general(tri, oh, (((1,), (0,)), ((), ())),
                        preferred_element_type=jnp.float32)
    counts = jnp.sum(oh, axis=0, keepdims=True)
    nb = jnp.floor((counts + (TB - 1)) / TB)
    e_io0 = lax.broadcasted_iota(jnp.int32, (E, E), 0)
    e_io1 = lax.broadcasted_iota(jnp.int32, (E, E), 1)
    triu = (e_io0 < e_io1).astype(jnp.float32)
    base_block = lax.dot_general(nb, triu, (((1,), (0,)), ((), ())),
                                 preferred_element_type=jnp.float32)
    slot_base = TB * base_block
    vals = slot_base + c
    dest0 = jnp.sum(oh1 * vals, axis=1, keepdims=True)
    dest1 = jnp.sum(oh2 * vals, axis=1, keepdims=True)
    d0_ref[...] = jnp.transpose(dest0.astype(jnp.int32)).reshape(T)
    d1_ref[...] = jnp.transpose(dest1.astype(jnp.int32)).reshape(T)
    wt0_ref[...] = jnp.transpose(w1).reshape(T)
    wt1_ref[...] = jnp.transpose(w2).reshape(T)

    ends = base_block + nb
    jrow = lax.broadcasted_iota(jnp.int32, (1, 128), 1).astype(jnp.float32)
    cmp = (jnp.transpose(base_block) <= jrow).astype(jnp.float32)
    bexp = jnp.sum(cmp, axis=0, keepdims=True) - 1.0
    num_active = jnp.sum(nb, axis=1, keepdims=True)
    lane = lax.broadcasted_iota(jnp.int32, (1, 128), 1)
    meta = jnp.where(lane == 127, num_active, bexp)
    be_ref[...] = meta.astype(jnp.int32).reshape(128)


def _k1(x, W_router):
    return pl.pallas_call(
        _k1_body,
        out_shape=(
            jax.ShapeDtypeStruct((T,), jnp.int32),
            jax.ShapeDtypeStruct((T,), jnp.int32),
            jax.ShapeDtypeStruct((T,), jnp.float32),
            jax.ShapeDtypeStruct((T,), jnp.float32),
            jax.ShapeDtypeStruct((128,), jnp.int32),
        ),
    )(x, W_router)


def _k2_body(x_hbm, d0_hbm, d1_hbm, xs_hbm,
             d0_v, d1_v, rows_v, sem, sem2):
    wid = lax.axis_index("s") * 2 + lax.axis_index("c")
    base = wid * TPT
    pltpu.sync_copy(d0_hbm.at[pl.ds(base, TPT)], d0_v)
    pltpu.sync_copy(d1_hbm.at[pl.ds(base, TPT)], d1_v)
    pltpu.sync_copy(x_hbm.at[pl.ds(base, TPT)], rows_v)
    c0 = pltpu.async_copy(rows_v, xs_hbm.at[d0_v], sem)
    c1 = pltpu.async_copy(rows_v, xs_hbm.at[d1_v], sem2)
    c0.wait()
    c1.wait()


def _k2(x, dest0, dest1):
    mesh = plsc.VectorSubcoreMesh(core_axis_name="c", subcore_axis_name="s")
    return pl.kernel(
        _k2_body,
        mesh=mesh,
        compiler_params=pltpu.CompilerParams(needs_layout_passes=False),
        out_type=jax.ShapeDtypeStruct((SLOTS, H), jnp.float32),
        scratch_types=[
            pltpu.VMEM((TPT,), jnp.int32),
            pltpu.VMEM((TPT,), jnp.int32),
            pltpu.VMEM((TPT, H), jnp.float32),
            pltpu.SemaphoreType.DMA,
            pltpu.SemaphoreType.DMA,
        ],
    )(x, dest0, dest1)


def _k3_body(be_ref, xs_ref, wg0_ref, wu0_ref, wd0_ref,
             wg1_ref, wu1_ref, wd1_ref, ys_ref):
    j = pl.program_id(0)
    na = be_ref[127]

    for p in range(2):
        @pl.when(2 * j + p < na)
        def _mlp():
            xs = xs_ref[pl.ds(p * TB, TB), :]
            wg = (wg0_ref, wg1_ref)[p]
            wu = (wu0_ref, wu1_ref)[p]
            wd = (wd0_ref, wd1_ref)[p]
            g = lax.dot_general(xs, wg[0], (((1,), (0,)), ((), ())),
                                preferred_element_type=jnp.float32)
            u = lax.dot_general(xs, wu[0], (((1,), (0,)), ((), ())),
                                preferred_element_type=jnp.float32)
            hmid = g * jax.nn.sigmoid(g) * u
            ys_ref[pl.ds(p * TB, TB), :] = lax.dot_general(
                hmid, wd[0], (((1,), (0,)), ((), ())),
                preferred_element_type=jnp.float32)


def _k3(xs, W_gate, W_up, W_down, be):
    def _be(idx):
        return lambda j, be: (be[jnp.minimum(idx(j), NB - 1)], 0, 0)

    grid_spec = pltpu.PrefetchScalarGridSpec(
        num_scalar_prefetch=1,
        grid=(NB2,),
        in_specs=[
            pl.BlockSpec((2 * TB, H), lambda j, be: (j, 0)),
            pl.BlockSpec((1, H, I), _be(lambda j: 2 * j)),
            pl.BlockSpec((1, H, I), _be(lambda j: 2 * j)),
            pl.BlockSpec((1, I, H), _be(lambda j: 2 * j)),
            pl.BlockSpec((1, H, I), _be(lambda j: 2 * j + 1)),
            pl.BlockSpec((1, H, I), _be(lambda j: 2 * j + 1)),
            pl.BlockSpec((1, I, H), _be(lambda j: 2 * j + 1)),
        ],
        out_specs=pl.BlockSpec((2 * TB, H), lambda j, be: (j, 0)),
    )
    return pl.pallas_call(
        _k3_body,
        grid_spec=grid_spec,
        out_shape=jax.ShapeDtypeStruct((SLOTS2, H), jnp.float32),
    )(be, xs, W_gate, W_up, W_down, W_gate, W_up, W_down)


def _k4_body(ys_hbm, d0_hbm, d1_hbm, wt0_hbm, wt1_hbm, out_hbm,
             d0_v, d1_v, w0_v, w1_v, rows0_v, rows1_v, sem, sem2):
    wid = lax.axis_index("s") * 2 + lax.axis_index("c")
    base = wid * TPT
    pltpu.sync_copy(d0_hbm.at[pl.ds(base, TPT)], d0_v)
    pltpu.sync_copy(d1_hbm.at[pl.ds(base, TPT)], d1_v)
    pltpu.sync_copy(wt0_hbm.at[pl.ds(base, TPT)], w0_v)
    pltpu.sync_copy(wt1_hbm.at[pl.ds(base, TPT)], w1_v)
    c0 = pltpu.async_copy(ys_hbm.at[d0_v], rows0_v, sem)
    c1 = pltpu.async_copy(ys_hbm.at[d1_v], rows1_v, sem2)
    c0.wait()
    c1.wait()

    def _tok(r, _):
        rsplat = jnp.full((16,), 0, jnp.int32) + r
        wa = plsc.load_gather(w0_v, [rsplat])
        wb = plsc.load_gather(w1_v, [rsplat])
        def _chunk(cidx, _):
            a = rows0_v[r, pl.ds(cidx * 16, 16)]
            b = rows1_v[r, pl.ds(cidx * 16, 16)]
            rows0_v[r, pl.ds(cidx * 16, 16)] = a * wa + b * wb
            return 0
        lax.fori_loop(0, H // 16, _chunk, 0, unroll=8)
        return 0
    lax.fori_loop(0, TPT, _tok, 0)
    pltpu.sync_copy(rows0_v, out_hbm.at[pl.ds(base, TPT)])


def _k4(ys, dest0, dest1, wt0, wt1):
    mesh = plsc.VectorSubcoreMesh(core_axis_name="c", subcore_axis_name="s")
    return pl.kernel(
        _k4_body,
        mesh=mesh,
        compiler_params=pltpu.CompilerParams(needs_layout_passes=False),
        out_type=jax.ShapeDtypeStruct((T, H), jnp.float32),
        scratch_types=[
            pltpu.VMEM((TPT,), jnp.int32),
            pltpu.VMEM((TPT,), jnp.int32),
            pltpu.VMEM((TPT,), jnp.float32),
            pltpu.VMEM((TPT,), jnp.float32),
            pltpu.VMEM((TPT, H), jnp.float32),
            pltpu.VMEM((TPT, H), jnp.float32),
            pltpu.SemaphoreType.DMA,
            pltpu.SemaphoreType.DMA,
        ],
    )(ys, dest0, dest1, wt0, wt1)


def kernel(hidden_states, W_router, W_gate, W_up, W_down):
    b, s, h = hidden_states.shape
    x = hidden_states.reshape(T, h)
    dest0, dest1, wt0, wt1, be = _k1(x, W_router)
    xs = _k2(x, dest0, dest1)
    ys = _k3(xs, W_gate, W_up, W_down, be)
    out = _k4(ys, dest0, dest1, wt0, wt1)
    return out.reshape(b, s, h)


# speedup vs baseline: 5.1685x; 1.7491x over previous
import jax
import jax.numpy as jnp
from jax.experimental import pallas as pl
from jax.experimental.pallas import tpu as pltpu

E = 16
TOP_K = 2
H = 768
I = 256
EPB = 2
NE = E // EPB
MB = 4


def _moe_body(x_ref, wr_ref, wg_ref, wu_ref, wd_ref, out_ref,
              rw_ref, hmid_ref, wdb_ref):
    j = pl.program_id(0)
    T = x_ref.shape[0]

    @pl.when(j == 0)
    def _router():
        x = x_ref[...]
        logits = jax.lax.dot_general(
            x, wr_ref[...], (((1,), (1,)), ((), ())),
            preferred_element_type=jnp.float32)
        scores = jax.nn.sigmoid(logits)
        eidx = jax.lax.broadcasted_iota(jnp.int32, scores.shape, 1)
        m1 = jnp.max(scores, axis=1, keepdims=True)
        is1 = scores == m1
        i1 = jnp.min(jnp.where(is1, eidx, E), axis=1, keepdims=True)
        excl = eidx == i1
        s2 = jnp.where(excl, -jnp.inf, scores)
        m2 = jnp.max(s2, axis=1, keepdims=True)
        i2 = jnp.min(jnp.where(s2 == m2, eidx, E), axis=1, keepdims=True)
        denom = m1 + m2 + 1e-20
        w1 = m1 / denom
        w2 = m2 / denom
        rw_ref[...] = (jnp.where(eidx == i1, w1, 0.0) +
                       jnp.where(eidx == i2, w2, 0.0))

    @pl.when(j < NE)
    def _experts():
        x = x_ref[...]
        eidx = jax.lax.broadcasted_iota(jnp.int32, (T, E), 1)
        hmids = []
        for p in range(EPB):
            e = j * EPB + p
            g = jax.lax.dot_general(x, wg_ref[p], (((1,), (0,)), ((), ())),
                                    preferred_element_type=jnp.float32)
            u = jax.lax.dot_general(x, wu_ref[p], (((1,), (0,)), ((), ())),
                                    preferred_element_type=jnp.float32)
            w_e = jnp.sum(jnp.where(eidx == e, rw_ref[...], 0.0),
                          axis=1, keepdims=True)
            hmids.append((g * jax.nn.sigmoid(g) * u * w_e).astype(jnp.bfloat16))
        wdb = wd_ref[...].astype(jnp.bfloat16)
        for k in range(NE):
            @pl.when(j == k)
            def _store():
                base = k * EPB * I
                for p in range(EPB):
                    hmid_ref[:, base + p * I:base + (p + 1) * I] = hmids[p]
                    wdb_ref[base + p * I:base + (p + 1) * I, :] = wdb[p]

    @pl.when(j >= NE)
    def _down():
        m = j - NE
        rows = T // MB
        hm = hmid_ref[pl.ds(m * rows, rows), :]
        out_ref[...] = jax.lax.dot_general(
            hm, wdb_ref[...], (((1,), (0,)), ((), ())),
            preferred_element_type=jnp.float32)


def kernel(hidden_states, W_router, W_gate, W_up, W_down):
    b, s, h = hidden_states.shape
    T = b * s
    x = hidden_states.reshape(T, h)
    out = pl.pallas_call(
        _moe_body,
        grid=(NE + MB,),
        in_specs=[
            pl.BlockSpec((T, H), lambda j: (0, 0)),
            pl.BlockSpec((E, H), lambda j: (0, 0)),
            pl.BlockSpec((EPB, H, I),
                         lambda j: (jnp.minimum(j, NE - 1), 0, 0)),
            pl.BlockSpec((EPB, H, I),
                         lambda j: (jnp.minimum(j, NE - 1), 0, 0)),
            pl.BlockSpec((EPB, I, H),
                         lambda j: (jnp.minimum(j, NE - 1), 0, 0)),
        ],
        out_specs=pl.BlockSpec(
            (T // MB, H), lambda j: (jnp.clip(j - NE, 0, MB - 1), 0)),
        out_shape=jax.ShapeDtypeStruct((T, H), jnp.float32),
        scratch_shapes=[
            pltpu.VMEM((T, E), jnp.float32),
            pltpu.VMEM((T, E * I), jnp.bfloat16),
            pltpu.VMEM((E * I, H), jnp.bfloat16),
        ],
    )(x, W_router, W_gate, W_up, W_down)
    return out.reshape(b, s, h)


# speedup vs baseline: 5.2075x; 1.0075x over previous
import jax
import jax.numpy as jnp
from jax.experimental import pallas as pl
from jax.experimental.pallas import tpu as pltpu

E = 16
TOP_K = 2
H = 768
I = 256
EPB = 2
NE = E // EPB
MB = 2


def _moe_body(x_ref, wr_ref, wg_ref, wu_ref, wd_ref, out_ref,
              rw_ref, hmid_ref, wdb_ref):
    j = pl.program_id(0)
    T = x_ref.shape[0]

    @pl.when(j == 0)
    def _router():
        x = x_ref[...]
        logits = jax.lax.dot_general(
            x, wr_ref[...], (((1,), (1,)), ((), ())),
            preferred_element_type=jnp.float32)
        scores = jax.nn.sigmoid(logits)
        eidx = jax.lax.broadcasted_iota(jnp.int32, scores.shape, 1)
        m1 = jnp.max(scores, axis=1, keepdims=True)
        is1 = scores == m1
        i1 = jnp.min(jnp.where(is1, eidx, E), axis=1, keepdims=True)
        excl = eidx == i1
        s2 = jnp.where(excl, -jnp.inf, scores)
        m2 = jnp.max(s2, axis=1, keepdims=True)
        i2 = jnp.min(jnp.where(s2 == m2, eidx, E), axis=1, keepdims=True)
        denom = m1 + m2 + 1e-20
        w1 = m1 / denom
        w2 = m2 / denom
        rw_ref[...] = (jnp.where(eidx == i1, w1, 0.0) +
                       jnp.where(eidx == i2, w2, 0.0))

    @pl.when(j < NE)
    def _experts():
        x = x_ref[...]
        eidx = jax.lax.broadcasted_iota(jnp.int32, (T, E), 1)
        hmids = []
        for p in range(EPB):
            e = j * EPB + p
            g = jax.lax.dot_general(x, wg_ref[p], (((1,), (0,)), ((), ())),
                                    preferred_element_type=jnp.float32)
            u = jax.lax.dot_general(x, wu_ref[p], (((1,), (0,)), ((), ())),
                                    preferred_element_type=jnp.float32)
            w_e = jnp.sum(jnp.where(eidx == e, rw_ref[...], 0.0),
                          axis=1, keepdims=True)
            hmids.append((g * jax.nn.sigmoid(g) * u * w_e).astype(jnp.bfloat16))
        wdb = wd_ref[...].astype(jnp.bfloat16)
        for k in range(NE):
            @pl.when(j == k)
            def _store():
                base = k * EPB * I
                for p in range(EPB):
                    hmid_ref[:, base + p * I:base + (p + 1) * I] = hmids[p]
                    wdb_ref[base + p * I:base + (p + 1) * I, :] = wdb[p]

    @pl.when(j >= NE)
    def _down():
        m = j - NE
        rows = T // MB
        hm = hmid_ref[pl.ds(m * rows, rows), :]
        out_ref[...] = jax.lax.dot_general(
            hm, wdb_ref[...], (((1,), (0,)), ((), ())),
            preferred_element_type=jnp.float32)


def kernel(hidden_states, W_router, W_gate, W_up, W_down):
    b, s, h = hidden_states.shape
    T = b * s
    x = hidden_states.reshape(T, h)
    out = pl.pallas_call(
        _moe_body,
        grid=(NE + MB,),
        in_specs=[
            pl.BlockSpec((T, H), lambda j: (0, 0)),
            pl.BlockSpec((E, H), lambda j: (0, 0)),
            pl.BlockSpec((EPB, H, I),
                         lambda j: (jnp.minimum(j, NE - 1), 0, 0)),
            pl.BlockSpec((EPB, H, I),
                         lambda j: (jnp.minimum(j, NE - 1), 0, 0)),
            pl.BlockSpec((EPB, I, H),
                         lambda j: (jnp.minimum(j, NE - 1), 0, 0)),
        ],
        out_specs=pl.BlockSpec(
            (T // MB, H), lambda j: (jnp.clip(j - NE, 0, MB - 1), 0)),
        out_shape=jax.ShapeDtypeStruct((T, H), jnp.float32),
        scratch_shapes=[
            pltpu.VMEM((T, E), jnp.float32),
            pltpu.VMEM((T, E * I), jnp.bfloat16),
            pltpu.VMEM((E * I, H), jnp.bfloat16),
        ],
    )(x, W_router, W_gate, W_up, W_down)
    return out.reshape(b, s, h)
